# baseline (speedup 1.0000x reference)
import functools

import jax
import jax.numpy as jnp
from jax import lax
from jax.experimental import pallas as pl
from jax.experimental.pallas import tpu as pltpu
from jax.experimental.pallas import tpu_sc as plsc


def _build_kernel(F, V, D, B):
    info = plsc.get_sparse_core_info()
    NC, NS = info.num_cores, info.num_subcores
    NW = NC * NS
    rows_per_w = B // NW
    CB = 16
    nchunk = rows_per_w // CB
    PC = CB * F
    NPV = PC // 16
    GCH = 104
    NG = PC // GCH
    KD = D // 16
    FD = F * D
    inv_n = 1.0 / float(FD)

    mesh = plsc.VectorSubcoreMesh(core_axis_name="c", subcore_axis_name="s")

    @functools.partial(
        pl.kernel,
        out_type=jax.ShapeDtypeStruct((B, FD), jnp.float32),
        mesh=mesh,
        compiler_params=pltpu.CompilerParams(use_tc_tiling_on_sc=True),
        scratch_types=[
            pltpu.VMEM((F, rows_per_w), jnp.int32),
            pltpu.VMEM((F, CB), jnp.int32),
            pltpu.VMEM((F, CB), jnp.int32),
            pltpu.VMEM((F, CB, 2 * D), jnp.float32),
            pltpu.VMEM((CB, FD), jnp.float32),
            pltpu.VMEM((FD,), jnp.float32),
            pltpu.VMEM((FD,), jnp.float32),
            pltpu.SemaphoreType.DMA,
        ],
    )
    def body(xt_hbm, tab_hbm, gamma_hbm, beta_hbm, out_hbm,
             idb_v, gp_v, hb_v, g2_v, st_v, gam_v, bet_v, sem):
        wid = lax.axis_index("s") * NC + lax.axis_index("c")
        pltpu.sync_copy(gamma_hbm, gam_v)
        pltpu.sync_copy(beta_hbm, bet_v)
        lane = lax.iota(jnp.int32, 16)
        perms = [lane ^ sh for sh in (8, 4, 2, 1)]
        gdn = lax.GatherDimensionNumbers(
            offset_dims=(), collapsed_slice_dims=(0,), start_index_map=(0,))

        def lane_total(v):
            for p in perms:
                v = v + lax.gather(
                    v, p[:, None], dimension_numbers=gdn, slice_sizes=(1,),
                    mode=lax.GatherScatterMode.PROMISE_IN_BOUNDS)
            return v

        pltpu.sync_copy(xt_hbm.at[:, pl.ds(wid * rows_per_w, rows_per_w)],
                        idb_v)

        def chunk_body(c, carry):
            base = (wid * nchunk + c) * CB

            def pidx_body(f, carry2):
                raw = idb_v[f, pl.ds(c * CB, CB)]
                raw = jnp.minimum(jnp.maximum(raw, 0), V - 1)
                gp_v[f, pl.ds(0, CB)] = (
                    f * (V // 2) + lax.shift_right_logical(raw, 1))
                hb_v[f, pl.ds(0, CB)] = lax.bitwise_and(raw, 1)
                return carry2

            lax.fori_loop(0, F, pidx_body, 0)

            def fire_f(f, carry2):
                pltpu.async_copy(
                    tab_hbm.at[gp_v.at[f]], g2_v.at[f], sem)
                return carry2

            lax.fori_loop(0, F, fire_f, 0)

            def drain_f(f, carry2):
                pltpu.make_async_copy(
                    tab_hbm.at[pl.ds(0, CB)], g2_v.at[f], sem).wait()
                return carry2

            lax.fori_loop(0, F, drain_f, 0)

            def comp_body(f, carry2):
                hv = hb_v[f, pl.ds(0, CB)]
                for j in range(CB):
                    off = hv[j] * D
                    for kk in range(KD):
                        st_v[j, pl.ds(f * D + kk * 16, 16)] = (
                            g2_v[f, j, pl.ds(off + kk * 16, 16)])
                return carry2

            lax.fori_loop(0, F, comp_body, 0)

            def row_body(j, carry2):
                def stat_body(f, sq):
                    s, q = sq
                    for k in range(KD):
                        v = st_v[j, pl.ds(f * D + k * 16, 16)]
                        s = s + v
                        q = q + v * v
                    return (s, q)

                zeros = jnp.zeros((16,), jnp.float32)
                s, q = lax.fori_loop(0, F, stat_body, (zeros, zeros))
                mean = lane_total(s) * inv_n
                var = lane_total(q) * inv_n - mean * mean
                av = var + 1e-5
                ii = lax.bitcast_convert_type(av, jnp.int32)
                ii = 0x5F3759DF - lax.shift_right_arithmetic(ii, 1)
                y = lax.bitcast_convert_type(ii, jnp.float32)
                y = y * (1.5 - 0.5 * av * y * y)
                y = y * (1.5 - 0.5 * av * y * y)
                y = y * (1.5 - 0.5 * av * y * y)
                c1 = y
                c0 = -mean * y

                def norm_body(f, carry3):
                    for k in range(KD):
                        sl = pl.ds(f * D + k * 16, 16)
                        v = st_v[j, sl]
                        t = v * c1 + c0
                        st_v[j, sl] = t * gam_v[sl] + bet_v[sl]
                    return carry3

                lax.fori_loop(0, F, norm_body, 0)
                return carry2

            lax.fori_loop(0, CB, row_body, 0)
            pltpu.sync_copy(st_v, out_hbm.at[pl.ds(base, CB)])
            return carry

        lax.fori_loop(0, nchunk, chunk_body, 0)

    return body


def kernel(x_cat, tables, gamma, beta):
    B, F = x_cat.shape
    _, V, D = tables.shape
    xt = x_cat.T
    tab2 = tables.reshape(F * V // 2, 2 * D)
    return _build_kernel(F, V, D, B)(xt, tab2, gamma, beta)


# speedup vs baseline: 1.3841x; 1.3841x over previous
import functools

import jax
import jax.numpy as jnp
from jax import lax
from jax.experimental import pallas as pl
from jax.experimental.pallas import tpu as pltpu
from jax.experimental.pallas import tpu_sc as plsc


def _build_kernel(F, V, D, B):
    info = plsc.get_sparse_core_info()
    NC, NS = info.num_cores, info.num_subcores
    NW = NC * NS
    rows_per_w = B // NW
    CB = 32
    nchunk = rows_per_w // CB
    NGV = CB // 16
    KD = D // 16
    FD = F * D
    inv_n = 1.0 / float(FD)

    mesh = plsc.VectorSubcoreMesh(core_axis_name="c", subcore_axis_name="s")

    @functools.partial(
        pl.kernel,
        out_type=jax.ShapeDtypeStruct((B, FD), jnp.float32),
        mesh=mesh,
        compiler_params=pltpu.CompilerParams(use_tc_tiling_on_sc=True),
        scratch_types=[
            pltpu.VMEM((F, rows_per_w), jnp.int32),
            pltpu.VMEM((CB * 8, D), jnp.float32),
            pltpu.VMEM((CB * 8, D), jnp.float32),
            pltpu.VMEM((CB, FD), jnp.float32),
            pltpu.VMEM((FD,), jnp.float32),
            pltpu.VMEM((FD,), jnp.float32),
            pltpu.SemaphoreType.DMA,
            pltpu.SemaphoreType.DMA,
        ],
    )
    def body(xt_hbm, tab_hbm, gamma_hbm, beta_hbm, out_hbm,
             idb_v, boxa_v, boxb_v, st_v, gam_v, bet_v, sema, semb):
        wid = lax.axis_index("s") * NC + lax.axis_index("c")
        pltpu.sync_copy(gamma_hbm, gam_v)
        pltpu.sync_copy(beta_hbm, bet_v)
        lane = lax.iota(jnp.int32, 16)
        perms = [lane ^ sh for sh in (8, 4, 2, 1)]
        gdn = lax.GatherDimensionNumbers(
            offset_dims=(), collapsed_slice_dims=(0,), start_index_map=(0,))

        def lane_total(v):
            for p in perms:
                v = v + lax.gather(
                    v, p[:, None], dimension_numbers=gdn, slice_sizes=(1,),
                    mode=lax.GatherScatterMode.PROMISE_IN_BOUNDS)
            return v

        pltpu.sync_copy(xt_hbm.at[:, pl.ds(wid * rows_per_w, rows_per_w)],
                        idb_v)

        def chunk_body(c, carry):
            base = (wid * nchunk + c) * CB

            def fire(f, box, sem):
                for g in range(NGV):
                    iv = idb_v[f, pl.ds(c * CB + g * 16, 16)]
                    iv = jnp.minimum(jnp.maximum(iv, 0), V - 1)
                    for k in range(16):
                        j = g * 16 + k
                        v0 = pl.multiple_of(
                            lax.bitwise_and(iv[k], jnp.int32(-8)), 8)
                        pltpu.async_copy(
                            tab_hbm.at[f, pl.ds(v0, 8), :],
                            box.at[pl.ds(j * 8, 8)], sem)

            def drain(box, sem):
                pltpu.make_async_copy(
                    tab_hbm.at[0, pl.ds(0, CB * 8), :], box, sem).wait()

            def compact(f, box):
                for g in range(NGV):
                    iv = idb_v[f, pl.ds(c * CB + g * 16, 16)]
                    iv = jnp.minimum(jnp.maximum(iv, 0), V - 1)
                    for k in range(16):
                        j = g * 16 + k
                        row = j * 8 + lax.bitwise_and(iv[k], jnp.int32(7))
                        for kk in range(KD):
                            st_v[j, pl.ds(f * D + kk * 16, 16)] = (
                                box[row, pl.ds(kk * 16, 16)])

            fire(0, boxa_v, sema)

            def field_pair(i, carry2):
                f0 = 2 * i
                fire(f0 + 1, boxb_v, semb)
                drain(boxa_v, sema)
                compact(f0, boxa_v)

                @pl.when(i < F // 2 - 1)
                def _():
                    fire(f0 + 2, boxa_v, sema)

                drain(boxb_v, semb)
                compact(f0 + 1, boxb_v)
                return carry2

            lax.fori_loop(0, F // 2, field_pair, 0)

            def row_body(j, carry2):
                def stat_body(f, sq):
                    s, q = sq
                    for k in range(KD):
                        v = st_v[j, pl.ds(f * D + k * 16, 16)]
                        s = s + v
                        q = q + v * v
                    return (s, q)

                zeros = jnp.zeros((16,), jnp.float32)
                s, q = lax.fori_loop(0, F, stat_body, (zeros, zeros))
                mean = lane_total(s) * inv_n
                var = lane_total(q) * inv_n - mean * mean
                av = var + 1e-5
                ii = lax.bitcast_convert_type(av, jnp.int32)
                ii = 0x5F3759DF - lax.shift_right_arithmetic(ii, 1)
                y = lax.bitcast_convert_type(ii, jnp.float32)
                y = y * (1.5 - 0.5 * av * y * y)
                y = y * (1.5 - 0.5 * av * y * y)
                y = y * (1.5 - 0.5 * av * y * y)
                c1 = y
                c0 = -mean * y

                def norm_body(f, carry3):
                    for k in range(KD):
                        sl = pl.ds(f * D + k * 16, 16)
                        v = st_v[j, sl]
                        t = v * c1 + c0
                        st_v[j, sl] = t * gam_v[sl] + bet_v[sl]
                    return carry3

                lax.fori_loop(0, F, norm_body, 0)
                return carry2

            lax.fori_loop(0, CB, row_body, 0)
            pltpu.sync_copy(st_v, out_hbm.at[pl.ds(base, CB)])
            return carry

        lax.fori_loop(0, nchunk, chunk_body, 0)

    return body


def kernel(x_cat, tables, gamma, beta):
    B, F = x_cat.shape
    _, V, D = tables.shape
    return _build_kernel(F, V, D, B)(x_cat.T, tables, gamma, beta)


# speedup vs baseline: 2.0786x; 1.5018x over previous
import functools

import jax
import jax.numpy as jnp
from jax import lax
from jax.experimental import pallas as pl
from jax.experimental.pallas import tpu as pltpu
from jax.experimental.pallas import tpu_sc as plsc


def _build_kernel(F, V, D, B):
    info = plsc.get_sparse_core_info()
    NC, NS = info.num_cores, info.num_subcores
    NW = NC * NS
    rows_per_w = B // NW
    CB = 32
    nchunk = rows_per_w // CB
    NGV = CB // 16
    KD = D // 16
    FD = F * D
    inv_n = 1.0 / float(FD)

    mesh = plsc.VectorSubcoreMesh(core_axis_name="c", subcore_axis_name="s")

    @functools.partial(
        pl.kernel,
        out_type=jax.ShapeDtypeStruct((B, FD), jnp.float32),
        mesh=mesh,
        compiler_params=pltpu.CompilerParams(use_tc_tiling_on_sc=True),
        scratch_types=[
            pltpu.VMEM((F, rows_per_w), jnp.int32),
            pltpu.VMEM((CB * 8, D), jnp.float32),
            pltpu.VMEM((CB * 8, D), jnp.float32),
            pltpu.VMEM((CB, FD), jnp.float32),
            pltpu.VMEM((FD,), jnp.float32),
            pltpu.VMEM((FD,), jnp.float32),
            pltpu.SemaphoreType.DMA,
            pltpu.SemaphoreType.DMA,
        ],
    )
    def body(xt_hbm, tab_hbm, gamma_hbm, beta_hbm, out_hbm,
             idb_v, boxa_v, boxb_v, st_v, gam_v, bet_v, sema, semb):
        wid = lax.axis_index("s") * NC + lax.axis_index("c")
        pltpu.sync_copy(gamma_hbm, gam_v)
        pltpu.sync_copy(beta_hbm, bet_v)
        lane = lax.iota(jnp.int32, 16)
        perms = [lane ^ sh for sh in (8, 4, 2, 1)]
        gdn = lax.GatherDimensionNumbers(
            offset_dims=(), collapsed_slice_dims=(0,), start_index_map=(0,))

        def lane_total(v):
            for p in perms:
                v = v + lax.gather(
                    v, p[:, None], dimension_numbers=gdn, slice_sizes=(1,),
                    mode=lax.GatherScatterMode.PROMISE_IN_BOUNDS)
            return v

        pltpu.sync_copy(xt_hbm.at[:, pl.ds(wid * rows_per_w, rows_per_w)],
                        idb_v)

        def chunk_body(c, carry):
            base = (wid * nchunk + c) * CB

            def fire(f, box, sem):
                for g in range(NGV):
                    iv = idb_v[f, pl.ds(c * CB + g * 16, 16)]
                    iv = jnp.minimum(jnp.maximum(iv, 0), V - 1)
                    for k in range(16):
                        j = g * 16 + k
                        v0 = pl.multiple_of(
                            f * V + lax.bitwise_and(iv[k], jnp.int32(-8)), 8)
                        pltpu.async_copy(
                            tab_hbm.at[pl.ds(v0, 8), :],
                            box.at[pl.ds(j * 8, 8)], sem)

            def drain(box, sem):
                pltpu.make_async_copy(
                    tab_hbm.at[pl.ds(0, CB * 8), :], box, sem).wait()

            def compact(f, box):
                for g in range(NGV):
                    iv = idb_v[f, pl.ds(c * CB + g * 16, 16)]
                    iv = jnp.minimum(jnp.maximum(iv, 0), V - 1)
                    for k in range(16):
                        j = g * 16 + k
                        row = j * 8 + lax.bitwise_and(iv[k], jnp.int32(7))
                        for kk in range(KD):
                            st_v[j, pl.ds(f * D + kk * 16, 16)] = (
                                box[row, pl.ds(kk * 16, 16)])

            fire(0, boxa_v, sema)

            def field_pair(i, carry2):
                f0 = 2 * i
                fire(f0 + 1, boxb_v, semb)
                drain(boxa_v, sema)
                compact(f0, boxa_v)

                @pl.when(i < F // 2 - 1)
                def _():
                    fire(f0 + 2, boxa_v, sema)

                drain(boxb_v, semb)
                compact(f0 + 1, boxb_v)
                return carry2

            lax.fori_loop(0, F // 2, field_pair, 0)

            def row_body(j, carry2):
                def stat_body(f, sq):
                    s, q = sq
                    for k in range(KD):
                        v = st_v[j, pl.ds(f * D + k * 16, 16)]
                        s = s + v
                        q = q + v * v
                    return (s, q)

                zeros = jnp.zeros((16,), jnp.float32)
                s, q = lax.fori_loop(0, F, stat_body, (zeros, zeros))
                mean = lane_total(s) * inv_n
                var = lane_total(q) * inv_n - mean * mean
                av = var + 1e-5
                ii = lax.bitcast_convert_type(av, jnp.int32)
                ii = 0x5F3759DF - lax.shift_right_arithmetic(ii, 1)
                y = lax.bitcast_convert_type(ii, jnp.float32)
                y = y * (1.5 - 0.5 * av * y * y)
                y = y * (1.5 - 0.5 * av * y * y)
                y = y * (1.5 - 0.5 * av * y * y)
                c1 = y
                c0 = -mean * y

                def norm_body(f, carry3):
                    for k in range(KD):
                        sl = pl.ds(f * D + k * 16, 16)
                        v = st_v[j, sl]
                        t = v * c1 + c0
                        st_v[j, sl] = t * gam_v[sl] + bet_v[sl]
                    return carry3

                lax.fori_loop(0, F, norm_body, 0)
                return carry2

            lax.fori_loop(0, CB, row_body, 0)
            pltpu.sync_copy(st_v, out_hbm.at[pl.ds(base, CB)])
            return carry

        lax.fori_loop(0, nchunk, chunk_body, 0)

    return body


def kernel(x_cat, tables, gamma, beta):
    B, F = x_cat.shape
    _, V, D = tables.shape
    tab2 = tables.reshape(F * V, D)
    return _build_kernel(F, V, D, B)(x_cat.T, tab2, gamma, beta)
